# baseline (speedup 1.0000x reference)
import functools

import jax
import jax.numpy as jnp
from jax import lax
from jax.experimental import pallas as pl
from jax.experimental.pallas import tpu as pltpu
from jax.experimental.pallas import tpu_sc as plsc

_NC = 2
_NS = 16
_NW = _NC * _NS
_L = 16


@functools.lru_cache(maxsize=None)
def _transpose_table(V: int, D: int):
    VB = 384
    NFULL = V // VB
    TAIL = V - NFULL * VB
    NT = NFULL // _NW
    NEXTRA = NFULL - NT * _NW
    PB = VB // 2
    mesh = plsc.VectorSubcoreMesh(core_axis_name="c", subcore_axis_name="s")

    @functools.partial(
        pl.kernel,
        mesh=mesh,
        compiler_params=pltpu.CompilerParams(needs_layout_passes=False),
        out_type=jax.ShapeDtypeStruct((V // 2, 2 * D), jnp.float32),
        scratch_types=[
            pltpu.VMEM((D, VB), jnp.float32),
            pltpu.VMEM((D, VB), jnp.float32),
            pltpu.VMEM((PB, 2 * D), jnp.float32),
            pltpu.VMEM((PB, 2 * D), jnp.float32),
            pltpu.VMEM((D, TAIL), jnp.float32),
            pltpu.SemaphoreType.DMA,
            pltpu.SemaphoreType.DMA,
            pltpu.SemaphoreType.DMA,
            pltpu.SemaphoreType.DMA,
        ],
    )
    def k(tt_hbm, tail_hbm, outp_hbm, tb0, tb1, tp0, tp1, tbt, si0, si1, so0, so1):
        wid = lax.axis_index("s") * _NC + lax.axis_index("c")
        lane = lax.iota(jnp.int32, _L)

        tbs = (tb0, tb1)
        tps = (tp0, tp1)
        sis = (si0, si1)
        sos = (so0, so1)

        def in_copy(blk, par):
            return pltpu.make_async_copy(
                tt_hbm.at[:, pl.ds(blk * VB, VB)], tbs[par], sis[par])

        def out_copy(blk, par):
            return pltpu.make_async_copy(
                tps[par], outp_hbm.at[pl.ds(blk * PB, PB)], sos[par])

        def transpose_block(tb, tp, npairs):
            nv = 2 * npairs
            vgs = []
            for vg in range(nv // _L):
                vb = lane + vg * _L
                vgs.append((vb,
                            lax.shift_right_logical(vb, 1),
                            lax.shift_left(lax.bitwise_and(vb, 1), 6)))

            def cbody(c, carry):
                cc = lax.bitwise_and(c + lane, D - 1)
                vals = [plsc.load_gather(tb, [cc, vb]) for vb, _, _ in vgs]
                for (vb, prow, jbase), val in zip(vgs, vals):
                    plsc.store_scatter(tp, [prow, jbase + cc], val)
                return carry

            lax.fori_loop(0, D, cbody, 0)

        in_copy(wid, 0).start()
        in_copy(wid + _NW, 1).start()

        def body(t, carry):
            for par in range(2):
                g = 2 * t + par
                blk = wid + _NW * g
                in_copy(blk, par).wait()

                @pl.when(g >= 2)
                def _():
                    out_copy(wid + _NW * (g - 2), par).wait()

                transpose_block(tbs[par], tps[par], PB)
                out_copy(blk, par).start()

                @pl.when(g + 2 < NT)
                def _():
                    in_copy(wid + _NW * (g + 2), par).start()

            return carry

        lax.fori_loop(0, NT // 2, body, 0)
        if NT % 2:
            gl = NT - 1
            blk = wid + _NW * gl
            in_copy(blk, 0).wait()
            out_copy(wid + _NW * (gl - 2), 0).wait()
            transpose_block(tb0, tp0, PB)
            out_copy(blk, 0).start()
            out_copy(wid + _NW * (gl - 1), 1).wait()
            out_copy(blk, 0).wait()
        else:
            out_copy(wid + _NW * (NT - 2), 0).wait()
            out_copy(wid + _NW * (NT - 1), 1).wait()

        @pl.when(wid < NEXTRA)
        def _():
            blk = NFULL - NEXTRA + wid
            in_copy(blk, 0).start()
            in_copy(blk, 0).wait()
            transpose_block(tb0, tp0, PB)
            out_copy(blk, 0).start()
            out_copy(blk, 0).wait()

        @pl.when(wid == NEXTRA)
        def _():
            pltpu.sync_copy(tail_hbm, tbt)
            transpose_block(tbt, tp0, TAIL // 2)
            pltpu.sync_copy(tp0.at[pl.ds(0, TAIL // 2)],
                            outp_hbm.at[pl.ds(NFULL * PB, TAIL // 2)])

    return k


@functools.lru_cache(maxsize=None)
def _embed_lookup(A: int, Bb: int, V: int, D: int):
    AW = A // _NW
    NG = AW // _L
    mesh = plsc.VectorSubcoreMesh(core_axis_name="c", subcore_axis_name="s")

    @functools.partial(
        pl.kernel,
        mesh=mesh,
        compiler_params=pltpu.CompilerParams(needs_layout_passes=False),
        out_type=jax.ShapeDtypeStruct((Bb, D, A), jnp.float32),
        scratch_types=[
            pltpu.VMEM((Bb, AW), jnp.int32),
            pltpu.VMEM((4, AW), jnp.int32),
            pltpu.VMEM((4, AW), jnp.int32),
            pltpu.VMEM((AW, 2 * D), jnp.float32),
            pltpu.VMEM((AW, 2 * D), jnp.float32),
            pltpu.VMEM((AW, 2 * D), jnp.float32),
            pltpu.VMEM((AW, 2 * D), jnp.float32),
            pltpu.VMEM((D, AW), jnp.float32),
            pltpu.VMEM((D, AW), jnp.float32),
            pltpu.SemaphoreType.DMA,
            pltpu.SemaphoreType.DMA,
            pltpu.SemaphoreType.DMA,
            pltpu.SemaphoreType.DMA,
            pltpu.SemaphoreType.DMA,
            pltpu.SemaphoreType.DMA,
        ],
    )
    def k(xt_hbm, tab_hbm, out_hbm, idx_v, pidx_v, off_v,
          g0, g1, g2, g3, tbuf0, tbuf1,
          sem0, sem1, sem2, sem3, wb0, wb1):
        wid = lax.axis_index("s") * _NC + lax.axis_index("c")
        a0 = wid * AW
        pltpu.sync_copy(xt_hbm.at[:, pl.ds(a0, AW)], idx_v)

        lane = lax.iota(jnp.int32, _L)

        gbufs = (g0, g1, g2, g3)
        sems = (sem0, sem1, sem2, sem3)
        tbufs = (tbuf0, tbuf1)
        wbs = (wb0, wb1)

        def prep(b, par):
            for kk in range(NG):
                v = idx_v[b, pl.ds(kk * _L, _L)]
                pidx_v[par, pl.ds(kk * _L, _L)] = lax.shift_right_logical(v, 1)
                off_v[par, pl.ds(kk * _L, _L)] = lax.shift_left(
                    lax.bitwise_and(v, 1), 6)

        def g_copy(par):
            return pltpu.make_async_copy(
                tab_hbm.at[pidx_v.at[par]], gbufs[par], sems[par])

        def wb_copy(b, tbuf, wbsem):
            return pltpu.make_async_copy(
                tbuf, out_hbm.at[b, :, pl.ds(a0, AW)], wbsem)

        def process(b, par):
            gbuf = gbufs[par]
            tbuf = tbufs[par % 2]
            groups = [(lane + kk * _L, off_v[par, pl.ds(kk * _L, _L)])
                      for kk in range(NG)]

            @pl.when(b >= 2)
            def _():
                wb_copy(b - 2, tbuf, wbs[par % 2]).wait()

            def cbody(c, carry):
                cc = lax.bitwise_and(c + lane, D - 1)
                vals = [plsc.load_gather(gbuf, [row, off + cc])
                        for row, off in groups]
                for (row, _), val in zip(groups, vals):
                    plsc.store_scatter(tbuf, [cc, row], val)
                return carry

            lax.fori_loop(0, D, cbody, 0)
            wb_copy(b, tbuf, wbs[par % 2]).start()

        for par in range(3):
            prep(par, par)
            g_copy(par).start()

        def body(q, carry):
            for par in range(4):
                b = 4 * q + par
                g_copy(par).wait()
                process(b, par)
                npar = (par + 3) % 4

                @pl.when(b + 3 < Bb)
                def _():
                    prep(b + 3, npar)
                    g_copy(npar).start()

            return carry

        lax.fori_loop(0, Bb // 4, body, 0)
        wb_copy(Bb - 2, tbuf0, wb0).wait()
        wb_copy(Bb - 1, tbuf1, wb1).wait()

    return k


def kernel(x, table):
    A, Bb = x.shape
    V, D = table.shape
    tt = table.T
    tab_pairs = _transpose_table(V, D)(tt, tt[:, V - V % 256:])
    out = _embed_lookup(A, Bb, V, D)(x.T, tab_pairs)
    return out.transpose(2, 0, 1)


# speedup vs baseline: 1.0358x; 1.0358x over previous
import functools

import jax
import jax.numpy as jnp
from jax import lax
from jax.experimental import pallas as pl
from jax.experimental.pallas import tpu as pltpu
from jax.experimental.pallas import tpu_sc as plsc

_NC = 2
_NS = 16
_NW = _NC * _NS
_L = 16


@functools.lru_cache(maxsize=None)
def _transpose_table(V: int, D: int):
    VB = 256
    NFULL = V // VB
    TAIL = V - NFULL * VB
    NT = NFULL // _NW
    NEXTRA = NFULL - NT * _NW
    PB = VB // 2
    mesh = plsc.VectorSubcoreMesh(core_axis_name="c", subcore_axis_name="s")

    @functools.partial(
        pl.kernel,
        mesh=mesh,
        compiler_params=pltpu.CompilerParams(needs_layout_passes=False),
        out_type=jax.ShapeDtypeStruct((V // 2, 2 * D), jnp.float32),
        scratch_types=[
            pltpu.VMEM((D, VB), jnp.float32),
            pltpu.VMEM((D, VB), jnp.float32),
            pltpu.VMEM((PB, 2 * D), jnp.float32),
            pltpu.VMEM((PB, 2 * D), jnp.float32),
            pltpu.VMEM((D, TAIL), jnp.float32),
            pltpu.SemaphoreType.DMA,
            pltpu.SemaphoreType.DMA,
            pltpu.SemaphoreType.DMA,
            pltpu.SemaphoreType.DMA,
        ],
    )
    def k(tt_hbm, tail_hbm, outp_hbm, tb0, tb1, tp0, tp1, tbt, si0, si1, so0, so1):
        wid = lax.axis_index("s") * _NC + lax.axis_index("c")
        lane = lax.iota(jnp.int32, _L)

        tbs = (tb0, tb1)
        tps = (tp0, tp1)
        sis = (si0, si1)
        sos = (so0, so1)

        def in_copy(blk, par):
            return pltpu.make_async_copy(
                tt_hbm.at[:, pl.ds(blk * VB, VB)], tbs[par], sis[par])

        def out_copy(blk, par):
            return pltpu.make_async_copy(
                tps[par], outp_hbm.at[pl.ds(blk * PB, PB)], sos[par])

        def transpose_block(tb, tp, npairs):
            nv = 2 * npairs
            vgs = []
            for vg in range(nv // _L):
                vb = lane + vg * _L
                vgs.append((vb,
                            lax.shift_right_logical(vb, 1),
                            lax.shift_left(lax.bitwise_and(vb, 1), 6)))

            def cbody(c, carry):
                cc = lax.bitwise_and(c + lane, D - 1)
                vals = [plsc.load_gather(tb, [cc, vb]) for vb, _, _ in vgs]
                for (vb, prow, jbase), val in zip(vgs, vals):
                    plsc.store_scatter(tp, [prow, jbase + cc], val)
                return carry

            lax.fori_loop(0, D, cbody, 0)

        in_copy(wid, 0).start()
        in_copy(wid + _NW, 1).start()

        def body(t, carry):
            for par in range(2):
                g = 2 * t + par
                blk = wid + _NW * g
                in_copy(blk, par).wait()

                @pl.when(g >= 2)
                def _():
                    out_copy(wid + _NW * (g - 2), par).wait()

                transpose_block(tbs[par], tps[par], PB)
                out_copy(blk, par).start()

                @pl.when(g + 2 < NT)
                def _():
                    in_copy(wid + _NW * (g + 2), par).start()

            return carry

        lax.fori_loop(0, NT // 2, body, 0)
        if NT % 2:
            gl = NT - 1
            blk = wid + _NW * gl
            in_copy(blk, 0).wait()
            out_copy(wid + _NW * (gl - 2), 0).wait()
            transpose_block(tb0, tp0, PB)
            out_copy(blk, 0).start()
            out_copy(wid + _NW * (gl - 1), 1).wait()
            out_copy(blk, 0).wait()
        else:
            out_copy(wid + _NW * (NT - 2), 0).wait()
            out_copy(wid + _NW * (NT - 1), 1).wait()

        @pl.when(wid < NEXTRA)
        def _():
            blk = NFULL - NEXTRA + wid
            in_copy(blk, 0).start()
            in_copy(blk, 0).wait()
            transpose_block(tb0, tp0, PB)
            out_copy(blk, 0).start()
            out_copy(blk, 0).wait()

        @pl.when(wid == NEXTRA)
        def _():
            pltpu.sync_copy(tail_hbm, tbt)
            transpose_block(tbt, tp0, TAIL // 2)
            pltpu.sync_copy(tp0.at[pl.ds(0, TAIL // 2)],
                            outp_hbm.at[pl.ds(NFULL * PB, TAIL // 2)])

    return k


@functools.lru_cache(maxsize=None)
def _embed_lookup(A: int, Bb: int, V: int, D: int):
    AW = A // _NW
    NG = AW // _L
    mesh = plsc.VectorSubcoreMesh(core_axis_name="c", subcore_axis_name="s")

    @functools.partial(
        pl.kernel,
        mesh=mesh,
        compiler_params=pltpu.CompilerParams(needs_layout_passes=False),
        out_type=jax.ShapeDtypeStruct((Bb, D, A), jnp.float32),
        scratch_types=[
            pltpu.VMEM((Bb, AW), jnp.int32),
            pltpu.VMEM((4, AW), jnp.int32),
            pltpu.VMEM((4, AW), jnp.int32),
            pltpu.VMEM((AW, 2 * D), jnp.float32),
            pltpu.VMEM((AW, 2 * D), jnp.float32),
            pltpu.VMEM((AW, 2 * D), jnp.float32),
            pltpu.VMEM((AW, 2 * D), jnp.float32),
            pltpu.VMEM((D, AW), jnp.float32),
            pltpu.VMEM((D, AW), jnp.float32),
            pltpu.SemaphoreType.DMA,
            pltpu.SemaphoreType.DMA,
            pltpu.SemaphoreType.DMA,
            pltpu.SemaphoreType.DMA,
            pltpu.SemaphoreType.DMA,
            pltpu.SemaphoreType.DMA,
        ],
    )
    def k(xt_hbm, tab_hbm, out_hbm, idx_v, pidx_v, off_v,
          g0, g1, g2, g3, tbuf0, tbuf1,
          sem0, sem1, sem2, sem3, wb0, wb1):
        wid = lax.axis_index("s") * _NC + lax.axis_index("c")
        a0 = wid * AW
        pltpu.sync_copy(xt_hbm.at[:, pl.ds(a0, AW)], idx_v)

        lane = lax.iota(jnp.int32, _L)

        gbufs = (g0, g1, g2, g3)
        sems = (sem0, sem1, sem2, sem3)
        tbufs = (tbuf0, tbuf1)
        wbs = (wb0, wb1)

        def prep(b, par):
            for kk in range(NG):
                v = idx_v[b, pl.ds(kk * _L, _L)]
                pidx_v[par, pl.ds(kk * _L, _L)] = lax.shift_right_logical(v, 1)
                off_v[par, pl.ds(kk * _L, _L)] = lax.shift_left(
                    lax.bitwise_and(v, 1), 6)

        def g_copy(par):
            return pltpu.make_async_copy(
                tab_hbm.at[pidx_v.at[par]], gbufs[par], sems[par])

        def wb_copy(b, tbuf, wbsem):
            return pltpu.make_async_copy(
                tbuf, out_hbm.at[b, :, pl.ds(a0, AW)], wbsem)

        def process(b, par):
            gbuf = gbufs[par]
            tbuf = tbufs[par % 2]
            groups = [(lane + kk * _L, off_v[par, pl.ds(kk * _L, _L)])
                      for kk in range(NG)]

            @pl.when(b >= 2)
            def _():
                wb_copy(b - 2, tbuf, wbs[par % 2]).wait()

            def cbody(c, carry):
                cc = lax.bitwise_and(c + lane, D - 1)
                vals = [plsc.load_gather(gbuf, [row, off + cc])
                        for row, off in groups]
                for (row, _), val in zip(groups, vals):
                    plsc.store_scatter(tbuf, [cc, row], val)
                return carry

            lax.fori_loop(0, D, cbody, 0)
            wb_copy(b, tbuf, wbs[par % 2]).start()

        for par in range(3):
            prep(par, par)
            g_copy(par).start()

        def body(q, carry):
            for par in range(4):
                b = 4 * q + par
                g_copy(par).wait()
                process(b, par)
                npar = (par + 3) % 4

                @pl.when(b + 3 < Bb)
                def _():
                    prep(b + 3, npar)
                    g_copy(npar).start()

            return carry

        lax.fori_loop(0, Bb // 4, body, 0)
        wb_copy(Bb - 2, tbuf0, wb0).wait()
        wb_copy(Bb - 1, tbuf1, wb1).wait()

    return k


def kernel(x, table):
    A, Bb = x.shape
    V, D = table.shape
    tt = table.T
    tab_pairs = _transpose_table(V, D)(tt, tt[:, V - V % 256:])
    out = _embed_lookup(A, Bb, V, D)(x.T, tab_pairs)
    return out.transpose(2, 0, 1)
